# baseline (speedup 1.0000x reference)
import functools
import math

import jax
import jax.numpy as jnp
from jax import lax
from jax.experimental import pallas as pl
from jax.experimental.pallas import tpu as pltpu
from jax.experimental.pallas import tpu_sc as plsc

D_MODEL = 64
SCALE = math.sqrt(D_MODEL)

_NC = 2
_NS = 16
_NW = _NC * _NS

N_BUF = 4
LEAD = 2
ROW_UNROLL = 4


def _make_kernel(R, S):
    assert R % _NW == 0
    r_per_w = R // _NW
    assert r_per_w % N_BUF == 0 and r_per_w > N_BUF
    assert S % ROW_UNROLL == 0 and S % 8 == 0

    mesh = plsc.VectorSubcoreMesh(core_axis_name="c", subcore_axis_name="s")

    @functools.partial(
        pl.kernel,
        mesh=mesh,
        out_type=jax.ShapeDtypeStruct((R, S, D_MODEL), jnp.float32),
        compiler_params=pltpu.CompilerParams(use_tc_tiling_on_sc=False),
        scratch_types=(
            [pltpu.VMEM((r_per_w, S), jnp.int32)]
            + [pltpu.VMEM((S, D_MODEL), jnp.float32) for _ in range(N_BUF)]
            + [pltpu.SemaphoreType.DMA for _ in range(2 * N_BUF)]
        ),
    )
    def emb(x_hbm, table_hbm, out_hbm, idx_all, *rest):
        rows = rest[:N_BUF]
        gsem = rest[N_BUF:2 * N_BUF]
        ssem = rest[2 * N_BUF:]

        wid = lax.axis_index("s") * _NC + lax.axis_index("c")
        rbase = wid * r_per_w

        def mk_gather(g, b):
            return pltpu.make_async_copy(
                table_hbm.at[idx_all.at[g]], rows[b], gsem[b])

        def mk_store(g, b):
            return pltpu.make_async_copy(
                rows[b], out_hbm.at[rbase + g], ssem[b])

        pltpu.sync_copy(x_hbm.at[pl.ds(rbase, r_per_w)], idx_all)
        for b in range(LEAD):
            mk_gather(b, b).start()

        def outer(i, carry):
            for b in range(N_BUF):
                g = i * N_BUF + b
                bb = (b + LEAD) % N_BUF

                @pl.when(g + LEAD < r_per_w)
                def _issue():
                    @pl.when(g >= LEAD)
                    def _drain():
                        mk_store(g - LEAD, bb).wait()
                    mk_gather(g + LEAD, bb).start()

                mk_gather(g, b).wait()

                buf = rows[b]

                def row_body(r, c):
                    for u in range(ROW_UNROLL):
                        rr = r * ROW_UNROLL + u
                        for c4 in range(D_MODEL // 16):
                            sl = pl.ds(c4 * 16, 16)
                            buf[rr, sl] = buf[rr, sl] * SCALE
                    return c

                lax.fori_loop(0, S // ROW_UNROLL, row_body, 0)
                mk_store(g, b).start()
            return carry

        lax.fori_loop(0, r_per_w // N_BUF, outer, 0)
        for g in range(r_per_w - N_BUF, r_per_w):
            mk_store(g, g % N_BUF).wait()

    return emb


def kernel(x, table):
    return _make_kernel(x.shape[0], x.shape[1])(x, table)
